# baseline (speedup 1.0000x reference)
import functools

import jax
import jax.numpy as jnp
from jax import lax
from jax.experimental import pallas as pl
from jax.experimental.pallas import tpu as pltpu
from jax.experimental.pallas import tpu_sc as plsc

N_FIELDS = 26
VOCAB = 100000
EMBED_DIM = 64
BATCH = 4096
OUT_D = N_FIELDS * EMBED_DIM

_NC = 2
_NS = 16
_NW = _NC * _NS
_BPW = BATCH // _NW
_CB = 8
_CROWS = _CB * N_FIELDS
_NCHUNK = _BPW // _CB
_LANES = 16
_VEC = EMBED_DIM // _LANES
_F2 = N_FIELDS - _LANES


@functools.partial(
    pl.kernel,
    out_type=jax.ShapeDtypeStruct((BATCH, OUT_D), jnp.float32),
    mesh=plsc.VectorSubcoreMesh(core_axis_name="c", subcore_axis_name="s"),
    scratch_types=[
        pltpu.VMEM((_BPW, N_FIELDS), jnp.int32),
        pltpu.VMEM((_CROWS // 8, 8, 2 * EMBED_DIM), jnp.float32),
        pltpu.VMEM((_CROWS // 8, 8, 2 * EMBED_DIM), jnp.float32),
        pltpu.VMEM((_CB, OUT_D), jnp.float32),
        pltpu.VMEM((_CB, OUT_D), jnp.float32),
        pltpu.VMEM((_CB, 2 * _LANES), jnp.int32),
        pltpu.VMEM((_CB, 2 * _LANES), jnp.int32),
        pltpu.SemaphoreType.DMA,
        pltpu.SemaphoreType.DMA,
        pltpu.SemaphoreType.DMA,
        pltpu.SemaphoreType.DMA,
    ],
    compiler_params=pltpu.CompilerParams(use_tc_tiling_on_sc=True),
)
def _gather(x_hbm, tab_hbm, out_hbm, xbuf, r0, r1, ob0, ob1, h0, h1,
            g0, g1, o0, o1):
    wid = lax.axis_index("s") * _NC + lax.axis_index("c")
    bbase = wid * _BPW
    rbufs = (r0, r1)
    obufs = (ob0, ob1)
    hbufs = (h0, h1)
    gsems = (g0, g1)
    osems = (o0, o1)
    lanes = lax.iota(jnp.int32, _LANES)
    fmul0 = lanes * (VOCAB // 16)
    fmul1 = (lanes + _F2) * (VOCAB // 16)

    pltpu.sync_copy(x_hbm.at[pl.ds(bbase, _BPW)], xbuf)

    def fire(m, b):
        def bb_body(bb, carry):
            row = m * _CB + bb
            v0 = xbuf[row, pl.ds(0, _LANES)]
            v1 = xbuf[row, pl.ds(_F2, _LANES)]
            t0 = lax.shift_right_logical(v0, 4) + fmul0
            t1 = lax.shift_right_logical(v1, 4) + fmul1
            s0 = lax.bitwise_and(lax.shift_right_logical(v0, 1), 7)
            s1 = lax.bitwise_and(lax.shift_right_logical(v1, 1), 7)
            hbufs[b][bb, pl.ds(0, _LANES)] = (
                lax.shift_left(lax.bitwise_and(v0, 1), 6))
            hbufs[b][bb, pl.ds(_LANES, _LANES)] = (
                lax.shift_left(lax.bitwise_and(v1, 1), 6))
            rb = bb * N_FIELDS
            for f in range(N_FIELDS):
                t = t0[f] if f < _LANES else t1[f - _F2]
                s = s0[f] if f < _LANES else s1[f - _F2]
                r = rb + f
                pltpu.make_async_copy(
                    tab_hbm.at[t, pl.ds(s, 1)],
                    rbufs[b].at[lax.shift_right_logical(r, 3),
                                pl.ds(lax.bitwise_and(r, 7), 1)],
                    gsems[b]).start()
            return carry

        lax.fori_loop(0, _CB, bb_body, 0)

    def gwait(b):
        pltpu.make_async_copy(
            tab_hbm.at[pl.ds(0, _CROWS // 8)], rbufs[b], gsems[b]).wait()

    def repack(b):
        def bb_body(bb, carry):
            hv0 = hbufs[b][bb, pl.ds(0, _LANES)]
            hv1 = hbufs[b][bb, pl.ds(_LANES, _LANES)]
            rb = bb * N_FIELDS
            for f in range(N_FIELDS):
                hoff = hv0[f] if f < _LANES else hv1[f - _F2]
                r = rb + f
                ri = lax.shift_right_logical(r, 3)
                rs = lax.bitwise_and(r, 7)
                for k in range(_VEC):
                    obufs[b][bb, pl.ds(f * EMBED_DIM + k * _LANES, _LANES)] = (
                        rbufs[b][ri, rs, pl.ds(hoff + k * _LANES, _LANES)])
            return carry

        lax.fori_loop(0, _CB, bb_body, 0)

    def ostart(m, b):
        pltpu.make_async_copy(
            obufs[b], out_hbm.at[pl.ds(bbase + m * _CB, _CB)],
            osems[b]).start()

    def owait(b):
        pltpu.make_async_copy(
            obufs[b], out_hbm.at[pl.ds(bbase, _CB)], osems[b]).wait()

    fire(0, 0)
    fire(1, 1)

    def pipe_body(i, carry):
        for b in range(2):
            m = 2 * i + b
            gwait(b)
            repack(b)
            ostart(m, b)
            owait(b)
            fire(m + 2, b)
        return carry

    lax.fori_loop(0, _NCHUNK // 2 - 1, pipe_body, 0)

    for m in (_NCHUNK - 2, _NCHUNK - 1):
        b = m % 2
        gwait(b)
        repack(b)
        ostart(m, b)
        owait(b)


def kernel(x, tables):
    tab = tables.reshape(N_FIELDS * VOCAB // 16, 8, 2 * EMBED_DIM)
    return _gather(x.astype(jnp.int32), tab)


# speedup vs baseline: 2.7736x; 2.7736x over previous
import functools

import jax
import jax.numpy as jnp
from jax import lax
from jax.experimental import pallas as pl
from jax.experimental.pallas import tpu as pltpu
from jax.experimental.pallas import tpu_sc as plsc

N_FIELDS = 26
VOCAB = 100000
EMBED_DIM = 64
BATCH = 4096
OUT_D = N_FIELDS * EMBED_DIM

_NC = 2
_NS = 16
_NW = _NC * _NS
_BPW = BATCH // _NW
_CB = 8
_CROWS = _CB * N_FIELDS
_NCHUNK = _BPW // _CB
_LANES = 16
_VEC = EMBED_DIM // _LANES


@functools.partial(
    pl.kernel,
    out_type=jax.ShapeDtypeStruct((BATCH, OUT_D), jnp.float32),
    mesh=plsc.VectorSubcoreMesh(core_axis_name="c", subcore_axis_name="s"),
    scratch_types=[
        pltpu.VMEM((_BPW, N_FIELDS), jnp.int32),
        pltpu.VMEM((_CROWS // 8, 8, EMBED_DIM), jnp.float32),
        pltpu.VMEM((_CROWS // 8, 8, EMBED_DIM), jnp.float32),
        pltpu.VMEM((_CB, OUT_D), jnp.float32),
        pltpu.VMEM((_CB, OUT_D), jnp.float32),
        pltpu.SemaphoreType.DMA,
        pltpu.SemaphoreType.DMA,
        pltpu.SemaphoreType.DMA,
        pltpu.SemaphoreType.DMA,
    ],
    compiler_params=pltpu.CompilerParams(use_tc_tiling_on_sc=True),
)
def _gather(x_hbm, tab_hbm, out_hbm, xbuf, r0, r1, ob0, ob1,
            g0, g1, o0, o1):
    wid = lax.axis_index("s") * _NC + lax.axis_index("c")
    bbase = wid * _BPW
    rbufs = (r0, r1)
    obufs = (ob0, ob1)
    gsems = (g0, g1)
    osems = (o0, o1)

    pltpu.sync_copy(x_hbm.at[pl.ds(bbase, _BPW)], xbuf)

    def fire(m, b):
        for bb in range(_CB):
            row = m * _CB + bb
            v0 = xbuf[row, pl.ds(0, _LANES)]
            v1 = xbuf[row, pl.ds(N_FIELDS - _LANES, _LANES)]
            for f in range(N_FIELDS):
                v = v0[f] if f < _LANES else v1[f - (N_FIELDS - _LANES)]
                t = lax.shift_right_logical(v, 3) + f * (VOCAB // 8)
                s = lax.bitwise_and(v, 7)
                r = bb * N_FIELDS + f
                pltpu.make_async_copy(
                    tab_hbm.at[t, pl.ds(s, 1)],
                    rbufs[b].at[r // 8, pl.ds(r % 8, 1)],
                    gsems[b]).start()

    def gwait(b):
        pltpu.make_async_copy(
            tab_hbm.at[pl.ds(0, _CROWS // 8)], rbufs[b], gsems[b]).wait()

    def repack(b):
        for bb in range(_CB):
            for f in range(N_FIELDS):
                r = bb * N_FIELDS + f
                for k in range(_VEC):
                    obufs[b][bb, pl.ds(f * EMBED_DIM + k * _LANES, _LANES)] = (
                        rbufs[b][r // 8, r % 8, pl.ds(k * _LANES, _LANES)])

    def ostart(m, b):
        pltpu.make_async_copy(
            obufs[b], out_hbm.at[pl.ds(bbase + m * _CB, _CB)],
            osems[b]).start()

    def owait(b):
        pltpu.make_async_copy(
            obufs[b], out_hbm.at[pl.ds(bbase, _CB)], osems[b]).wait()

    fire(0, 0)
    fire(1, 1)

    def pipe_body(i, carry):
        for b in range(2):
            m = 2 * i + b
            gwait(b)
            repack(b)
            ostart(m, b)
            owait(b)
            fire(m + 2, b)
        return carry

    lax.fori_loop(0, _NCHUNK // 2 - 1, pipe_body, 0)

    for m in (_NCHUNK - 2, _NCHUNK - 1):
        b = m % 2
        gwait(b)
        repack(b)
        ostart(m, b)
        owait(b)


def kernel(x, tables):
    tab = tables.reshape(N_FIELDS * VOCAB // 8, 8, EMBED_DIM)
    return _gather(x.astype(jnp.int32), tab)


# speedup vs baseline: 2.7970x; 1.0084x over previous
import functools

import jax
import jax.numpy as jnp
from jax import lax
from jax.experimental import pallas as pl
from jax.experimental.pallas import tpu as pltpu
from jax.experimental.pallas import tpu_sc as plsc

N_FIELDS = 26
VOCAB = 100000
EMBED_DIM = 64
BATCH = 4096
OUT_D = N_FIELDS * EMBED_DIM

_NC = 2
_NS = 16
_NW = _NC * _NS
_BPW = BATCH // _NW
_CB = 8
_CROWS = _CB * N_FIELDS
_NCHUNK = _BPW // _CB
_LANES = 16
_VEC = EMBED_DIM // _LANES


@functools.partial(
    pl.kernel,
    out_type=jax.ShapeDtypeStruct((BATCH, OUT_D), jnp.float32),
    mesh=plsc.VectorSubcoreMesh(core_axis_name="c", subcore_axis_name="s"),
    scratch_types=[
        pltpu.VMEM((_BPW, N_FIELDS), jnp.int32),
        pltpu.VMEM((_CROWS // 8, 8, EMBED_DIM), jnp.float32),
        pltpu.VMEM((_CROWS // 8, 8, EMBED_DIM), jnp.float32),
        pltpu.VMEM((_CB, OUT_D), jnp.float32),
        pltpu.VMEM((_CB, OUT_D), jnp.float32),
        pltpu.SemaphoreType.DMA,
        pltpu.SemaphoreType.DMA,
        pltpu.SemaphoreType.DMA,
        pltpu.SemaphoreType.DMA,
    ],
    compiler_params=pltpu.CompilerParams(use_tc_tiling_on_sc=True),
)
def _gather(x_hbm, tab_hbm, out_hbm, xbuf, r0, r1, ob0, ob1,
            g0, g1, o0, o1):
    wid = lax.axis_index("s") * _NC + lax.axis_index("c")
    bbase = wid * _BPW
    rbufs = (r0, r1)
    obufs = (ob0, ob1)
    gsems = (g0, g1)
    osems = (o0, o1)

    pltpu.sync_copy(x_hbm.at[pl.ds(bbase, _BPW)], xbuf)

    def fire(m, b):
        for bb in range(_CB):
            row = m * _CB + bb
            v0 = xbuf[row, pl.ds(0, _LANES)]
            v1 = xbuf[row, pl.ds(N_FIELDS - _LANES, _LANES)]
            for f in range(N_FIELDS):
                v = v0[f] if f < _LANES else v1[f - (N_FIELDS - _LANES)]
                t = lax.shift_right_logical(v, 3) + f * (VOCAB // 8)
                s = lax.bitwise_and(v, 7)
                r = bb * N_FIELDS + f
                pltpu.make_async_copy(
                    tab_hbm.at[t, pl.ds(s, 1)],
                    rbufs[b].at[r // 8, pl.ds(r % 8, 1)],
                    gsems[b]).start()

    def gwait(b):
        pltpu.make_async_copy(
            tab_hbm.at[pl.ds(0, _CROWS // 8)], rbufs[b], gsems[b]).wait()

    def repack(b):
        for bb in range(_CB):
            for f in range(N_FIELDS):
                r = bb * N_FIELDS + f
                for k in range(_VEC):
                    obufs[b][bb, pl.ds(f * EMBED_DIM + k * _LANES, _LANES)] = (
                        rbufs[b][r // 8, r % 8, pl.ds(k * _LANES, _LANES)])

    def ostart(m, b):
        pltpu.make_async_copy(
            obufs[b], out_hbm.at[pl.ds(bbase + m * _CB, _CB)],
            osems[b]).start()

    def owait(b):
        pltpu.make_async_copy(
            obufs[b], out_hbm.at[pl.ds(bbase, _CB)], osems[b]).wait()

    fire(0, 0)
    fire(1, 1)
    for b in range(2):
        pltpu.make_async_copy(
            out_hbm.at[pl.ds(bbase, _CB)], obufs[b], osems[b]).start()

    def pipe_body(i, carry):
        for b in range(2):
            m = 2 * i + b
            gwait(b)
            owait(b)
            repack(b)
            ostart(m, b)
            fire(m + 2, b)
        return carry

    lax.fori_loop(0, _NCHUNK // 2 - 1, pipe_body, 0)

    for m in (_NCHUNK - 2, _NCHUNK - 1):
        b = m % 2
        gwait(b)
        owait(b)
        repack(b)
        ostart(m, b)
    owait(0)
    owait(1)


def kernel(x, tables):
    tab = tables.reshape(N_FIELDS * VOCAB // 8, 8, EMBED_DIM)
    return _gather(x.astype(jnp.int32), tab)


# speedup vs baseline: 2.7986x; 1.0006x over previous
import functools

import jax
import jax.numpy as jnp
from jax import lax
from jax.experimental import pallas as pl
from jax.experimental.pallas import tpu as pltpu
from jax.experimental.pallas import tpu_sc as plsc

N_FIELDS = 26
VOCAB = 100000
EMBED_DIM = 64
BATCH = 4096
OUT_D = N_FIELDS * EMBED_DIM

_NC = 2
_NS = 16
_NW = _NC * _NS
_BPW = BATCH // _NW
_CB = 8
_CROWS = _CB * N_FIELDS
_NCHUNK = _BPW // _CB
_LANES = 16
_VEC = EMBED_DIM // _LANES


@functools.partial(
    pl.kernel,
    out_type=jax.ShapeDtypeStruct((BATCH, OUT_D), jnp.float32),
    mesh=plsc.VectorSubcoreMesh(core_axis_name="c", subcore_axis_name="s"),
    scratch_types=[
        pltpu.VMEM((_BPW, N_FIELDS), jnp.int32),
        pltpu.VMEM((_CROWS, EMBED_DIM), jnp.float32),
        pltpu.VMEM((_CROWS, EMBED_DIM), jnp.float32),
        pltpu.VMEM((_CB, OUT_D), jnp.float32),
        pltpu.VMEM((_CB, OUT_D), jnp.float32),
        pltpu.SemaphoreType.DMA,
        pltpu.SemaphoreType.DMA,
        pltpu.SemaphoreType.DMA,
        pltpu.SemaphoreType.DMA,
    ],
    compiler_params=pltpu.CompilerParams(use_tc_tiling_on_sc=True),
)
def _gather(x_hbm, tab_hbm, out_hbm, xbuf, r0, r1, ob0, ob1,
            g0, g1, o0, o1):
    wid = lax.axis_index("s") * _NC + lax.axis_index("c")
    bbase = wid * _BPW
    rbufs = (r0, r1)
    obufs = (ob0, ob1)
    gsems = (g0, g1)
    osems = (o0, o1)

    pltpu.sync_copy(x_hbm.at[pl.ds(bbase, _BPW)], xbuf)

    def fire(m, b):
        for bb in range(_CB):
            row = m * _CB + bb
            v0 = xbuf[row, pl.ds(0, _LANES)]
            v1 = xbuf[row, pl.ds(N_FIELDS - _LANES, _LANES)]
            for f in range(N_FIELDS):
                v = v0[f] if f < _LANES else v1[f - (N_FIELDS - _LANES)]
                r = bb * N_FIELDS + f
                pltpu.make_async_copy(
                    tab_hbm.at[pl.ds(v + f * VOCAB, 1)],
                    rbufs[b].at[pl.ds(r, 1)],
                    gsems[b]).start()

    def gwait(b):
        pltpu.make_async_copy(
            tab_hbm.at[pl.ds(0, _CROWS)], rbufs[b], gsems[b]).wait()

    def repack(b):
        for bb in range(_CB):
            for f in range(N_FIELDS):
                r = bb * N_FIELDS + f
                for k in range(_VEC):
                    obufs[b][bb, pl.ds(f * EMBED_DIM + k * _LANES, _LANES)] = (
                        rbufs[b][r, pl.ds(k * _LANES, _LANES)])

    def ostart(m, b):
        pltpu.make_async_copy(
            obufs[b], out_hbm.at[pl.ds(bbase + m * _CB, _CB)],
            osems[b]).start()

    def owait(b):
        pltpu.make_async_copy(
            obufs[b], out_hbm.at[pl.ds(bbase, _CB)], osems[b]).wait()

    fire(0, 0)
    fire(1, 1)
    for b in range(2):
        pltpu.make_async_copy(
            out_hbm.at[pl.ds(bbase, _CB)], obufs[b], osems[b]).start()

    def pipe_body(i, carry):
        for b in range(2):
            m = 2 * i + b
            gwait(b)
            owait(b)
            repack(b)
            ostart(m, b)
            fire(m + 2, b)
        return carry

    lax.fori_loop(0, _NCHUNK // 2 - 1, pipe_body, 0)

    for m in (_NCHUNK - 2, _NCHUNK - 1):
        b = m % 2
        gwait(b)
        owait(b)
        repack(b)
        ostart(m, b)
    owait(0)
    owait(1)


def kernel(x, tables):
    tab = tables.reshape(N_FIELDS * VOCAB, EMBED_DIM)
    return _gather(x.astype(jnp.int32), tab)
